# baseline (speedup 1.0000x reference)
import jax
import jax.numpy as jnp
from jax import lax
from jax.experimental import pallas as pl
from jax.experimental.pallas import tpu as pltpu

N_TOK = 128
D_IN = 2048
D_HID = 32768
K = 64
BH = 1024
NB = D_HID // BH
CW = 512
NC = D_HID // CW
T_LAD = 16
LAD_LO = 0x3F800000
LAD_STEP = (0x40A00000 - LAD_LO) // T_LAD
LAD_THR = [LAD_LO + k * LAD_STEP for k in range(T_LAD)]


def _lanes_fold_any(wide):
    nw = wide.shape[1] // 128
    parts = [wide[:, j * 128:(j + 1) * 128] for j in range(nw)]
    out = parts[0]
    for p in parts[1:]:
        out = out + p
    return out


def _lanes_fold(wide):
    return _lanes_fold_any(wide)


def _count_gt(h_ref, mid):
    acc = jnp.zeros((N_TOK, 128), jnp.int32)
    for c in range(NC):
        blk = h_ref[:, c * CW:(c + 1) * CW]
        bits = lax.bitcast_convert_type(blk, jnp.int32)
        acc = acc + _lanes_fold((bits > mid).astype(jnp.int32))
    return jnp.sum(acc, axis=1, keepdims=True)


def _count_eq_lt(h_ref, eq_bits, m):
    def body(c, acc):
        blk = h_ref[:, pl.ds(c * CW, CW)]
        bits = lax.bitcast_convert_type(blk, jnp.int32)
        idx = lax.broadcasted_iota(jnp.int32, (N_TOK, CW), 1) + c * CW
        hit = (bits == eq_bits) & (idx < m)
        return acc + _lanes_fold(hit.astype(jnp.int32))
    acc = lax.fori_loop(0, NC, body, jnp.zeros((N_TOK, 128), jnp.int32))
    return jnp.sum(acc, axis=1, keepdims=True)


def _select(hs_ref, rmax_ref, lad_ref, gt_ref, eq_ref, m_ref):
    rmax = jnp.max(rmax_ref[...], axis=1, keepdims=True)
    lo0 = jnp.full((N_TOK, 1), -1, jnp.int32)
    hi0 = rmax
    thr0 = jnp.full((N_TOK, 1), -1, jnp.int32)
    cnts = [jnp.sum(lad_ref[:, k * 128:(k + 1) * 128], axis=1,
                    keepdims=True) for k in range(T_LAD)]
    for k in range(T_LAD):
        tk = jnp.int32(LAD_THR[k])
        lo0 = jnp.where(cnts[k] >= K, tk, lo0)
        thr0 = jnp.where(cnts[k] == K, tk, thr0)
    for k in range(T_LAD - 1, -1, -1):
        tk = jnp.int32(LAD_THR[k])
        hi0 = jnp.where(cnts[k] < K, jnp.minimum(tk, hi0), hi0)

    def cond(st):
        lo, hi, thr, it = st
        active = jnp.logical_and(thr < 0, hi - lo > 1)
        return jnp.logical_and(it < 48, jnp.any(active))

    def body(st):
        lo, hi, thr, it = st
        active = jnp.logical_and(thr < 0, hi - lo > 1)
        shift = jnp.minimum(22 + it, 30)
        mid_h = jnp.clip(hi0 - (jnp.int32(1) << shift), lo + 1, hi - 1)
        mid_std = lo + ((hi - lo) >> 1)
        use_h = jnp.logical_and(lo < 0, it < 9)
        mid = jnp.where(use_h, mid_h, mid_std)
        c = _count_gt(hs_ref, mid)
        hit = jnp.logical_and(active, c == K)
        thr = jnp.where(hit, mid, thr)
        lo = jnp.where(jnp.logical_and(active, c >= K), mid, lo)
        hi = jnp.where(jnp.logical_and(active, c < K), mid, hi)
        return lo, hi, thr, it + 1

    lo, hi, thr, _ = lax.while_loop(
        cond, body, (lo0, hi0, thr0, jnp.int32(0)))
    done = thr >= 0

    any_tie = jnp.any(~done)

    def tie_path(_):
        c_hi = _count_gt(hs_ref, hi)
        r = K - c_hi
        mlo0 = jnp.zeros((N_TOK, 1), jnp.int32)
        mhi0 = jnp.full((N_TOK, 1), D_HID, jnp.int32)

        def mbody(_, st):
            mlo, mhi = st
            mmid = mlo + ((mhi - mlo) >> 1)
            cm = _count_eq_lt(hs_ref, hi, mmid)
            ge = cm >= r
            return jnp.where(ge, mlo, mmid), jnp.where(ge, mmid, mhi)

        mlo, mhi = lax.fori_loop(0, 15, mbody, (mlo0, mhi0))
        return mhi

    m_tie = lax.cond(any_tie, tie_path,
                     lambda _: jnp.zeros((N_TOK, 1), jnp.int32),
                     operand=None)

    gt_ref[...] = jnp.where(done, thr, hi)
    eq_ref[...] = jnp.where(done, jnp.full((N_TOK, 1), -1, jnp.int32), hi)
    m_ref[...] = jnp.where(done, jnp.zeros((N_TOK, 1), jnp.int32), m_tie)


def _fused_kernel(x_ref, we_ref, be_ref, wd_ref, bd_ref,
                  hsp_ref, dec_ref, stats_ref,
                  hs_ref, rmax_ref, lad_ref, gt_ref, eq_ref, m_ref, acc_ref,
                  l1_ref, l0_ref):
    i = pl.program_id(0)

    @pl.when(i < NB)
    def _encode():
        h_blk = lax.dot_general(x_ref[...], we_ref[...],
                                (((1,), (1,)), ((), ())),
                                preferred_element_type=jnp.float32)
        h_blk = jnp.maximum(h_blk + be_ref[...], 0.0)
        hs_ref[:, pl.ds(i * BH, BH)] = h_blk
        bits = lax.bitcast_convert_type(h_blk, jnp.int32)
        parts = [bits[:, j * 128:(j + 1) * 128] for j in range(BH // 128)]
        w = parts[0]
        for p in parts[1:]:
            w = jnp.maximum(w, p)

        lads = []
        for k in range(T_LAD):
            tk = jnp.int32(LAD_THR[k])
            cw = _lanes_fold_any((bits > tk).astype(jnp.int32))
            lads.append(cw)

        @pl.when(i == 0)
        def _():
            rmax_ref[...] = w
            for k in range(T_LAD):
                lad_ref[:, k * 128:(k + 1) * 128] = lads[k]

        @pl.when(i > 0)
        def _():
            rmax_ref[...] = jnp.maximum(rmax_ref[...], w)
            for k in range(T_LAD):
                lad_ref[:, k * 128:(k + 1) * 128] = (
                    lad_ref[:, k * 128:(k + 1) * 128] + lads[k])

    @pl.when(i == NB - 1)
    def _do_select():
        _select(hs_ref, rmax_ref, lad_ref, gt_ref, eq_ref, m_ref)

    @pl.when(i >= NB)
    def _decode():
        j = i - NB
        h_blk = hs_ref[:, pl.ds(j * BH, BH)]
        bits = lax.bitcast_convert_type(h_blk, jnp.int32)
        idx = lax.broadcasted_iota(jnp.int32, (N_TOK, BH), 1) + j * BH
        mask = jnp.logical_or(
            bits > gt_ref[...],
            jnp.logical_and(bits == eq_ref[...], idx < m_ref[...]))
        hs = jnp.where(mask, h_blk, 0.0)
        hsp_ref[...] = hs

        part = lax.dot_general(hs, wd_ref[...], (((1,), (1,)), ((), ())),
                               preferred_element_type=jnp.float32)
        l1p = jnp.sum(hs, axis=1, keepdims=True)
        l0p = jnp.sum((hs > 0.0).astype(jnp.float32), axis=1, keepdims=True)

        @pl.when(j == 0)
        def _():
            acc_ref[...] = part
            l1_ref[...] = l1p
            l0_ref[...] = l0p

        @pl.when(j > 0)
        def _():
            acc_ref[...] += part
            l1_ref[...] += l1p
            l0_ref[...] += l0p

        @pl.when(j == NB - 1)
        def _final():
            decoded = acc_ref[...] + bd_ref[...]
            dec_ref[...] = decoded
            d = decoded - x_ref[...]
            recon = jnp.sum(d * d, axis=1, keepdims=True)
            stats_ref[0:1, :] = jnp.sum(recon, axis=0, keepdims=True)
            stats_ref[1:2, :] = jnp.sum(l1_ref[...], axis=0, keepdims=True)
            stats_ref[2:3, :] = jnp.sum(l0_ref[...], axis=0, keepdims=True)


@jax.jit
def kernel(x, W_enc, b_enc, W_dec, b_dec):
    b_enc2 = b_enc.reshape(1, D_HID)
    b_dec2 = b_dec.reshape(1, D_IN)

    h_sparse, decoded, stats = pl.pallas_call(
        _fused_kernel,
        grid=(2 * NB,),
        in_specs=[
            pl.BlockSpec((N_TOK, D_IN), lambda i: (0, 0)),
            pl.BlockSpec((BH, D_IN), lambda i: (jnp.minimum(i, NB - 1), 0)),
            pl.BlockSpec((1, BH), lambda i: (0, jnp.minimum(i, NB - 1))),
            pl.BlockSpec((D_IN, BH), lambda i: (0, jnp.maximum(i - NB, 0))),
            pl.BlockSpec((1, D_IN), lambda i: (0, 0)),
        ],
        out_specs=[
            pl.BlockSpec((N_TOK, BH), lambda i: (0, jnp.maximum(i - NB, 0))),
            pl.BlockSpec((N_TOK, D_IN), lambda i: (0, 0)),
            pl.BlockSpec((8, 1), lambda i: (0, 0)),
        ],
        out_shape=[
            jax.ShapeDtypeStruct((N_TOK, D_HID), jnp.float32),
            jax.ShapeDtypeStruct((N_TOK, D_IN), jnp.float32),
            jax.ShapeDtypeStruct((8, 1), jnp.float32),
        ],
        scratch_shapes=[
            pltpu.VMEM((N_TOK, D_HID), jnp.float32),
            pltpu.VMEM((N_TOK, 128), jnp.int32),
            pltpu.VMEM((N_TOK, T_LAD * 128), jnp.int32),
            pltpu.VMEM((N_TOK, 1), jnp.int32),
            pltpu.VMEM((N_TOK, 1), jnp.int32),
            pltpu.VMEM((N_TOK, 1), jnp.int32),
            pltpu.VMEM((N_TOK, D_IN), jnp.float32),
            pltpu.VMEM((N_TOK, 1), jnp.float32),
            pltpu.VMEM((N_TOK, 1), jnp.float32),
        ],
        compiler_params=pltpu.CompilerParams(
            dimension_semantics=("arbitrary",)),
    )(x, W_enc, b_enc2, W_dec, b_dec2)

    recon_loss = stats[0, 0] / (N_TOK * D_IN)
    l1_loss = stats[1, 0] / (N_TOK * D_HID)
    l0_loss = stats[2, 0] / (N_TOK * D_HID)
    return (decoded, h_sparse, recon_loss, recon_loss, l1_loss, l0_loss)


# speedup vs baseline: 1.0437x; 1.0437x over previous
import jax
import jax.numpy as jnp
from jax import lax
from jax.experimental import pallas as pl
from jax.experimental.pallas import tpu as pltpu

N_TOK = 128
D_IN = 2048
D_HID = 32768
K = 64
BH = 1024
NB = D_HID // BH
CW = 512
NC = D_HID // CW


def _lanes_fold_any(wide):
    nw = wide.shape[1] // 128
    parts = [wide[:, j * 128:(j + 1) * 128] for j in range(nw)]
    out = parts[0]
    for p in parts[1:]:
        out = out + p
    return out


def _lanes_fold(wide):
    return _lanes_fold_any(wide)


def _count_gt(h_ref, mid):
    acc = jnp.zeros((N_TOK, 128), jnp.int32)
    for c in range(NC):
        blk = h_ref[:, c * CW:(c + 1) * CW]
        bits = lax.bitcast_convert_type(blk, jnp.int32)
        acc = acc + _lanes_fold((bits > mid).astype(jnp.int32))
    return jnp.sum(acc, axis=1, keepdims=True)


def _count_eq_lt(h_ref, eq_bits, m):
    def body(c, acc):
        blk = h_ref[:, pl.ds(c * CW, CW)]
        bits = lax.bitcast_convert_type(blk, jnp.int32)
        idx = lax.broadcasted_iota(jnp.int32, (N_TOK, CW), 1) + c * CW
        hit = (bits == eq_bits) & (idx < m)
        return acc + _lanes_fold(hit.astype(jnp.int32))
    acc = lax.fori_loop(0, NC, body, jnp.zeros((N_TOK, 128), jnp.int32))
    return jnp.sum(acc, axis=1, keepdims=True)


def _select(hs_ref, rmax_ref, gt_ref, eq_ref, m_ref):
    hi0 = jnp.max(rmax_ref[...], axis=1, keepdims=True)
    lo0 = jnp.full((N_TOK, 1), -1, jnp.int32)
    thr0 = jnp.full((N_TOK, 1), -1, jnp.int32)

    def cond(st):
        lo, hi, thr, it = st
        active = jnp.logical_and(thr < 0, hi - lo > 1)
        return jnp.logical_and(it < 48, jnp.any(active))

    def body(st):
        lo, hi, thr, it = st
        active = jnp.logical_and(thr < 0, hi - lo > 1)
        shift = jnp.minimum(22 + it, 30)
        mid_h = jnp.clip(hi0 - (jnp.int32(1) << shift), lo + 1, hi - 1)
        mid_std = lo + ((hi - lo) >> 1)
        use_h = jnp.logical_and(lo < 0, it < 9)
        mid = jnp.where(use_h, mid_h, mid_std)
        c = _count_gt(hs_ref, mid)
        hit = jnp.logical_and(active, c == K)
        thr = jnp.where(hit, mid, thr)
        lo = jnp.where(jnp.logical_and(active, c >= K), mid, lo)
        hi = jnp.where(jnp.logical_and(active, c < K), mid, hi)
        return lo, hi, thr, it + 1

    lo, hi, thr, _ = lax.while_loop(
        cond, body, (lo0, hi0, thr0, jnp.int32(0)))
    done = thr >= 0

    any_tie = jnp.any(~done)

    def tie_path(_):
        c_hi = _count_gt(hs_ref, hi)
        r = K - c_hi
        mlo0 = jnp.zeros((N_TOK, 1), jnp.int32)
        mhi0 = jnp.full((N_TOK, 1), D_HID, jnp.int32)

        def mbody(_, st):
            mlo, mhi = st
            mmid = mlo + ((mhi - mlo) >> 1)
            cm = _count_eq_lt(hs_ref, hi, mmid)
            ge = cm >= r
            return jnp.where(ge, mlo, mmid), jnp.where(ge, mmid, mhi)

        mlo, mhi = lax.fori_loop(0, 15, mbody, (mlo0, mhi0))
        return mhi

    m_tie = lax.cond(any_tie, tie_path,
                     lambda _: jnp.zeros((N_TOK, 1), jnp.int32),
                     operand=None)

    gt_ref[...] = jnp.where(done, thr, hi)
    eq_ref[...] = jnp.where(done, jnp.full((N_TOK, 1), -1, jnp.int32), hi)
    m_ref[...] = jnp.where(done, jnp.zeros((N_TOK, 1), jnp.int32), m_tie)


def _fused_kernel(x_ref, we_ref, be_ref, wd_ref, bd_ref,
                  hsp_ref, dec_ref, stats_ref,
                  hs_ref, rmax_ref, gt_ref, eq_ref, m_ref, acc_ref,
                  l1_ref, l0_ref):
    i = pl.program_id(0)

    @pl.when(i < NB)
    def _encode():
        h_blk = lax.dot_general(x_ref[...], we_ref[...],
                                (((1,), (1,)), ((), ())),
                                preferred_element_type=jnp.float32)
        h_blk = jnp.maximum(h_blk + be_ref[...], 0.0)
        hs_ref[:, pl.ds(i * BH, BH)] = h_blk
        bits = lax.bitcast_convert_type(h_blk, jnp.int32)
        parts = [bits[:, j * 128:(j + 1) * 128] for j in range(BH // 128)]
        w = parts[0]
        for p in parts[1:]:
            w = jnp.maximum(w, p)

        @pl.when(i == 0)
        def _():
            rmax_ref[...] = w

        @pl.when(i > 0)
        def _():
            rmax_ref[...] = jnp.maximum(rmax_ref[...], w)

    @pl.when(i == NB - 1)
    def _do_select():
        _select(hs_ref, rmax_ref, gt_ref, eq_ref, m_ref)

    @pl.when(i >= NB)
    def _decode():
        j = i - NB
        h_blk = hs_ref[:, pl.ds(j * BH, BH)]
        bits = lax.bitcast_convert_type(h_blk, jnp.int32)
        idx = lax.broadcasted_iota(jnp.int32, (N_TOK, BH), 1) + j * BH
        mask = jnp.logical_or(
            bits > gt_ref[...],
            jnp.logical_and(bits == eq_ref[...], idx < m_ref[...]))
        hs = jnp.where(mask, h_blk, 0.0)
        hsp_ref[...] = hs

        part = lax.dot_general(hs, wd_ref[...], (((1,), (1,)), ((), ())),
                               preferred_element_type=jnp.float32)
        l1p = jnp.sum(hs, axis=1, keepdims=True)
        l0p = jnp.sum((hs > 0.0).astype(jnp.float32), axis=1, keepdims=True)

        @pl.when(j == 0)
        def _():
            acc_ref[...] = part
            l1_ref[...] = l1p
            l0_ref[...] = l0p

        @pl.when(j > 0)
        def _():
            acc_ref[...] += part
            l1_ref[...] += l1p
            l0_ref[...] += l0p

        @pl.when(j == NB - 1)
        def _final():
            decoded = acc_ref[...] + bd_ref[...]
            dec_ref[...] = decoded
            d = decoded - x_ref[...]
            recon = jnp.sum(d * d, axis=1, keepdims=True)
            stats_ref[0:1, :] = jnp.sum(recon, axis=0, keepdims=True)
            stats_ref[1:2, :] = jnp.sum(l1_ref[...], axis=0, keepdims=True)
            stats_ref[2:3, :] = jnp.sum(l0_ref[...], axis=0, keepdims=True)


@jax.jit
def kernel(x, W_enc, b_enc, W_dec, b_dec):
    b_enc2 = b_enc.reshape(1, D_HID)
    b_dec2 = b_dec.reshape(1, D_IN)

    h_sparse, decoded, stats = pl.pallas_call(
        _fused_kernel,
        grid=(2 * NB,),
        in_specs=[
            pl.BlockSpec((N_TOK, D_IN), lambda i: (0, 0)),
            pl.BlockSpec((BH, D_IN), lambda i: (jnp.minimum(i, NB - 1), 0)),
            pl.BlockSpec((1, BH), lambda i: (0, jnp.minimum(i, NB - 1))),
            pl.BlockSpec((D_IN, BH), lambda i: (0, jnp.maximum(i - NB, 0))),
            pl.BlockSpec((1, D_IN), lambda i: (0, 0)),
        ],
        out_specs=[
            pl.BlockSpec((N_TOK, BH), lambda i: (0, jnp.maximum(i - NB, 0))),
            pl.BlockSpec((N_TOK, D_IN), lambda i: (0, 0)),
            pl.BlockSpec((8, 1), lambda i: (0, 0)),
        ],
        out_shape=[
            jax.ShapeDtypeStruct((N_TOK, D_HID), jnp.float32),
            jax.ShapeDtypeStruct((N_TOK, D_IN), jnp.float32),
            jax.ShapeDtypeStruct((8, 1), jnp.float32),
        ],
        scratch_shapes=[
            pltpu.VMEM((N_TOK, D_HID), jnp.float32),
            pltpu.VMEM((N_TOK, 128), jnp.int32),
            pltpu.VMEM((N_TOK, 1), jnp.int32),
            pltpu.VMEM((N_TOK, 1), jnp.int32),
            pltpu.VMEM((N_TOK, 1), jnp.int32),
            pltpu.VMEM((N_TOK, D_IN), jnp.float32),
            pltpu.VMEM((N_TOK, 1), jnp.float32),
            pltpu.VMEM((N_TOK, 1), jnp.float32),
        ],
        compiler_params=pltpu.CompilerParams(
            dimension_semantics=("arbitrary",)),
    )(x, W_enc, b_enc2, W_dec, b_dec2)

    recon_loss = stats[0, 0] / (N_TOK * D_IN)
    l1_loss = stats[1, 0] / (N_TOK * D_HID)
    l0_loss = stats[2, 0] / (N_TOK * D_HID)
    return (decoded, h_sparse, recon_loss, recon_loss, l1_loss, l0_loss)
